# baseline (speedup 1.0000x reference)
import functools

import jax
import jax.numpy as jnp
from jax import lax
from jax.experimental import pallas as pl
from jax.experimental.pallas import tpu as pltpu
from jax.experimental.pallas import tpu_sc as plsc

N_EXPERTS = 16
LBL_COEF = 0.01

BLOCK_T = 1024
SC_LANES = 16
SC_WORKERS = 32


def _logits_kernel(x_ref, w_ref, logits_ref, loss_ref, acc_ref,
                   *, n_steps, n_tokens):
    step = pl.program_id(0)

    @pl.when(step == 0)
    def _init():
        acc_ref[...] = jnp.zeros_like(acc_ref)

    logits = jax.lax.dot_general(
        w_ref[...], x_ref[...],
        dimension_numbers=(((1,), (1,)), ((), ())),
        preferred_element_type=jnp.float32,
    )
    logits_ref[...] = logits

    m1 = jnp.max(logits, axis=0, keepdims=True)
    ex = jnp.exp(logits - m1)
    rp = ex / jnp.sum(ex, axis=0, keepdims=True)
    acc_ref[...] += jnp.sum(rp, axis=1, keepdims=True)

    @pl.when(step == n_steps - 1)
    def _finish():
        ep = acc_ref[...] / jnp.float32(n_tokens)
        loss_ref[0, 0] = LBL_COEF * jnp.sum(ep * jnp.log(ep + 1e-8))


def _route_rows(rows):
    minf = jnp.full((SC_LANES,), -jnp.inf, jnp.float32)
    m1 = rows[0]
    i1 = jnp.zeros((SC_LANES,), jnp.int32)
    for e in range(1, N_EXPERTS):
        g = rows[e] > m1
        m1 = jnp.where(g, rows[e], m1)
        i1 = jnp.where(g, jnp.int32(e), i1)
    m2 = minf
    i2 = jnp.zeros((SC_LANES,), jnp.int32)
    for e in range(N_EXPERTS):
        r = jnp.where(i1 == jnp.int32(e), minf, rows[e])
        g = r > m2
        m2 = jnp.where(g, r, m2)
        i2 = jnp.where(g, jnp.int32(e), i2)
    e2 = jnp.exp(m2 - m1)
    den = 1.0 + e2
    return 1.0 / den, e2 / den, i1, i2


def _sc_router(logits):
    n_tokens = logits.shape[1]
    per_w = n_tokens // SC_WORKERS
    mesh = plsc.VectorSubcoreMesh(core_axis_name="c", subcore_axis_name="s")

    @functools.partial(
        pl.kernel,
        mesh=mesh,
        out_type=[
            jax.ShapeDtypeStruct((2, n_tokens), jnp.float32),
            jax.ShapeDtypeStruct((2, n_tokens), jnp.int32),
        ],
        scratch_types=[
            pltpu.VMEM((N_EXPERTS, per_w), jnp.float32),
            pltpu.VMEM((2, per_w), jnp.float32),
            pltpu.VMEM((2, per_w), jnp.int32),
        ],
    )
    def router(l_hbm, p_hbm, i_hbm, l_v, p_v, i_v):
        wid = lax.axis_index("s") * 2 + lax.axis_index("c")
        base = wid * per_w
        pltpu.sync_copy(l_hbm.at[:, pl.ds(base, per_w)], l_v)

        @pl.loop(0, per_w, step=SC_LANES)
        def _(c):
            rows = [l_v[e, pl.ds(c, SC_LANES)] for e in range(N_EXPERTS)]
            p1, p2, i1, i2 = _route_rows(rows)
            p_v[0, pl.ds(c, SC_LANES)] = p1
            p_v[1, pl.ds(c, SC_LANES)] = p2
            i_v[0, pl.ds(c, SC_LANES)] = i1
            i_v[1, pl.ds(c, SC_LANES)] = i2

        pltpu.sync_copy(p_v, p_hbm.at[:, pl.ds(base, per_w)])
        pltpu.sync_copy(i_v, i_hbm.at[:, pl.ds(base, per_w)])

    return router(logits)


def kernel(x, W):
    b, s, d = x.shape
    n_tokens = b * s
    xf = x.reshape(n_tokens, d)
    n_steps = n_tokens // BLOCK_T

    logits, loss = pl.pallas_call(
        functools.partial(_logits_kernel, n_steps=n_steps, n_tokens=n_tokens),
        grid=(n_steps,),
        in_specs=[
            pl.BlockSpec((BLOCK_T, d), lambda i: (i, 0)),
            pl.BlockSpec((N_EXPERTS, d), lambda i: (0, 0)),
        ],
        out_specs=[
            pl.BlockSpec((N_EXPERTS, BLOCK_T), lambda i: (0, i)),
            pl.BlockSpec(memory_space=pltpu.SMEM),
        ],
        out_shape=[
            jax.ShapeDtypeStruct((N_EXPERTS, n_tokens), jnp.float32),
            jax.ShapeDtypeStruct((1, 1), jnp.float32),
        ],
        scratch_shapes=[pltpu.VMEM((N_EXPERTS, 1), jnp.float32)],
    )(xf, W)

    probs, idx = _sc_router(logits)
    return (probs.T.reshape(b, s, 2), idx.T.reshape(b, s, 2), loss[0, 0])


# speedup vs baseline: 1.0489x; 1.0489x over previous
import functools

import jax
import jax.numpy as jnp
from jax.experimental import pallas as pl
from jax.experimental.pallas import tpu as pltpu

N_EXPERTS = 16
LBL_COEF = 0.01

BLOCK_T = 1024


def _router_kernel(x_ref, w_ref, probs_ref, idx_ref, loss_ref, acc_ref,
                   *, n_steps, n_tokens):
    step = pl.program_id(0)

    @pl.when(step == 0)
    def _init():
        acc_ref[...] = jnp.zeros_like(acc_ref)

    x_blk = x_ref[...]
    w = w_ref[...]
    logits = jax.lax.dot_general(
        w, x_blk,
        dimension_numbers=(((1,), (1,)), ((), ())),
        preferred_element_type=jnp.float32,
    )

    row = jax.lax.broadcasted_iota(jnp.int32, logits.shape, 0)
    big = jnp.int32(N_EXPERTS)

    m1 = jnp.max(logits, axis=0, keepdims=True)
    i1 = jnp.min(jnp.where(logits == m1, row, big), axis=0, keepdims=True)
    masked = jnp.where(row == i1, -jnp.inf, logits)
    m2 = jnp.max(masked, axis=0, keepdims=True)
    i2 = jnp.min(jnp.where(masked == m2, row, big), axis=0, keepdims=True)

    e2 = jnp.exp(m2 - m1)
    denom = 1.0 + e2
    p1 = 1.0 / denom
    p2 = e2 / denom

    probs_ref[...] = jnp.concatenate([p1, p2], axis=0).T
    idx_ref[...] = jnp.concatenate([i1, i2], axis=0).T

    ex = jnp.exp(logits - m1)
    rp = ex / jnp.sum(ex, axis=0, keepdims=True)
    acc_ref[...] += jnp.sum(rp, axis=1, keepdims=True)

    @pl.when(step == n_steps - 1)
    def _finish():
        ep = acc_ref[...] / jnp.float32(n_tokens)
        loss_ref[0, 0] = LBL_COEF * jnp.sum(ep * jnp.log(ep + 1e-8))


def kernel(x, W):
    b, s, d = x.shape
    n_tokens = b * s
    xf = x.reshape(n_tokens, d)
    n_steps = n_tokens // BLOCK_T

    probs, idx, loss = pl.pallas_call(
        functools.partial(_router_kernel, n_steps=n_steps, n_tokens=n_tokens),
        grid=(n_steps,),
        in_specs=[
            pl.BlockSpec((BLOCK_T, d), lambda i: (i, 0)),
            pl.BlockSpec((N_EXPERTS, d), lambda i: (0, 0)),
        ],
        out_specs=[
            pl.BlockSpec((BLOCK_T, 2), lambda i: (i, 0)),
            pl.BlockSpec((BLOCK_T, 2), lambda i: (i, 0)),
            pl.BlockSpec(memory_space=pltpu.SMEM),
        ],
        out_shape=[
            jax.ShapeDtypeStruct((n_tokens, 2), jnp.float32),
            jax.ShapeDtypeStruct((n_tokens, 2), jnp.int32),
            jax.ShapeDtypeStruct((1, 1), jnp.float32),
        ],
        scratch_shapes=[pltpu.VMEM((N_EXPERTS, 1), jnp.float32)],
    )(xf, W)

    return (probs.reshape(b, s, 2), idx.reshape(b, s, 2), loss[0, 0])


# speedup vs baseline: 1.4564x; 1.3886x over previous
import functools

import jax
import jax.numpy as jnp
from jax.experimental import pallas as pl
from jax.experimental.pallas import tpu as pltpu

N_EXPERTS = 16
LBL_COEF = 0.01

BLOCK_T = 1024


def _router_kernel(x_ref, w_ref, probs_ref, idx_ref, loss_ref, acc_ref,
                   *, n_steps, n_tokens):
    step = pl.program_id(0)

    @pl.when(step == 0)
    def _init():
        acc_ref[...] = jnp.zeros_like(acc_ref)

    x_blk = x_ref[...]
    w = w_ref[...]
    logits = jax.lax.dot_general(
        w, x_blk,
        dimension_numbers=(((1,), (1,)), ((), ())),
        preferred_element_type=jnp.float32,
    )

    row = jax.lax.broadcasted_iota(jnp.int32, logits.shape, 0)
    big = jnp.int32(N_EXPERTS)

    m1 = jnp.max(logits, axis=0, keepdims=True)
    i1 = jnp.min(jnp.where(logits == m1, row, big), axis=0, keepdims=True)
    masked = jnp.where(row == i1, -jnp.inf, logits)
    m2 = jnp.max(masked, axis=0, keepdims=True)
    i2 = jnp.min(jnp.where(masked == m2, row, big), axis=0, keepdims=True)

    e2 = jnp.exp(m2 - m1)
    denom = 1.0 + e2
    p1 = 1.0 / denom
    p2 = e2 / denom

    probs_ref[...] = jnp.concatenate([p1, p2], axis=0)
    idx_ref[...] = jnp.concatenate([i1, i2], axis=0)

    ex = jnp.exp(logits - m1)
    rp = ex / jnp.sum(ex, axis=0, keepdims=True)
    acc_ref[...] += jnp.sum(rp, axis=1, keepdims=True)

    @pl.when(step == n_steps - 1)
    def _finish():
        ep = acc_ref[...] / jnp.float32(n_tokens)
        loss_ref[0, 0] = LBL_COEF * jnp.sum(ep * jnp.log(ep + 1e-8))


def kernel(x, W):
    b, s, d = x.shape
    n_tokens = b * s
    xf = x.reshape(n_tokens, d)
    n_steps = n_tokens // BLOCK_T

    probs, idx, loss = pl.pallas_call(
        functools.partial(_router_kernel, n_steps=n_steps, n_tokens=n_tokens),
        grid=(n_steps,),
        in_specs=[
            pl.BlockSpec((BLOCK_T, d), lambda i: (i, 0)),
            pl.BlockSpec((N_EXPERTS, d), lambda i: (0, 0)),
        ],
        out_specs=[
            pl.BlockSpec((2, BLOCK_T), lambda i: (0, i)),
            pl.BlockSpec((2, BLOCK_T), lambda i: (0, i)),
            pl.BlockSpec(memory_space=pltpu.SMEM),
        ],
        out_shape=[
            jax.ShapeDtypeStruct((2, n_tokens), jnp.float32),
            jax.ShapeDtypeStruct((2, n_tokens), jnp.int32),
            jax.ShapeDtypeStruct((1, 1), jnp.float32),
        ],
        scratch_shapes=[pltpu.VMEM((N_EXPERTS, 1), jnp.float32)],
    )(xf, W)

    return (probs.T.reshape(b, s, 2), idx.T.reshape(b, s, 2), loss[0, 0])
